# baseline (speedup 1.0000x reference)
import jax
import jax.numpy as jnp
from jax.experimental import pallas as pl
from jax.experimental.pallas import tpu as pltpu


def _body(x_ref, t_ref, h_ref, w_ref, o_ref, pos_ref):
    b = pl.program_id(1)

    @pl.when(b == 0)
    def _build_pos():
        t = t_ref[0]
        h = h_ref[...]
        w = w_ref[...]
        pos_ref[...] = h[:, None, :] + (w + t)[None, :, :]

    o_ref[0, 0] = x_ref[0, 0] + pos_ref[...]


def kernel(x, T_embed, H_embed, W_embed):
    B, T, H, W, C = x.shape
    return pl.pallas_call(
        _body,
        grid=(T, B),
        in_specs=[
            pl.BlockSpec((1, 1, H, W, C), lambda t, b: (b, t, 0, 0, 0)),
            pl.BlockSpec((1, 1, C), lambda t, b: (t, 0, 0)),
            pl.BlockSpec((H, C), lambda t, b: (0, 0)),
            pl.BlockSpec((W, C), lambda t, b: (0, 0)),
        ],
        out_specs=pl.BlockSpec((1, 1, H, W, C), lambda t, b: (b, t, 0, 0, 0)),
        out_shape=jax.ShapeDtypeStruct(x.shape, x.dtype),
        scratch_shapes=[pltpu.VMEM((H, W, C), x.dtype)],
        compiler_params=pltpu.CompilerParams(
            dimension_semantics=("arbitrary", "arbitrary"),
        ),
    )(x, T_embed[:T].reshape(T, 1, C), H_embed[:H], W_embed[:W])


# speedup vs baseline: 1.0916x; 1.0916x over previous
import jax
import jax.numpy as jnp
from jax.experimental import pallas as pl
from jax.experimental.pallas import tpu as pltpu


def _body(x_ref, t_ref, h_ref, w_ref, o_ref, pos_ref):
    b = pl.program_id(1)

    @pl.when(b == 0)
    def _build_pos():
        t = t_ref[0]
        h = h_ref[...]
        w = w_ref[...]
        pos_ref[...] = h[:, None, :] + (w + t)[None, :, :]

    o_ref[:, 0] = x_ref[:, 0] + pos_ref[...][None]


def kernel(x, T_embed, H_embed, W_embed):
    B, T, H, W, C = x.shape
    BB = 4
    return pl.pallas_call(
        _body,
        grid=(T, B // BB),
        in_specs=[
            pl.BlockSpec((BB, 1, H, W, C), lambda t, b: (b, t, 0, 0, 0)),
            pl.BlockSpec((1, 1, C), lambda t, b: (t, 0, 0)),
            pl.BlockSpec((H, C), lambda t, b: (0, 0)),
            pl.BlockSpec((W, C), lambda t, b: (0, 0)),
        ],
        out_specs=pl.BlockSpec((BB, 1, H, W, C), lambda t, b: (b, t, 0, 0, 0)),
        out_shape=jax.ShapeDtypeStruct(x.shape, x.dtype),
        scratch_shapes=[pltpu.VMEM((H, W, C), x.dtype)],
        compiler_params=pltpu.CompilerParams(
            dimension_semantics=("arbitrary", "arbitrary"),
        ),
    )(x, T_embed[:T].reshape(T, 1, C), H_embed[:H], W_embed[:W])


# speedup vs baseline: 1.1100x; 1.0168x over previous
import jax
import jax.numpy as jnp
from jax.experimental import pallas as pl
from jax.experimental.pallas import tpu as pltpu

_HB = 16


def _body(x_ref, t_ref, h_ref, w_ref, o_ref):
    t = t_ref[...]
    h = h_ref[...]
    w = w_ref[...]
    hw = h[:, None, :] + w[None, :, :]
    o_ref[0] = (x_ref[0] + t[:, None, None, :]) + hw[None, :, :, :]


def kernel(x, T_embed, H_embed, W_embed):
    B, T, H, W, C = x.shape
    return pl.pallas_call(
        _body,
        grid=(B, H // _HB),
        in_specs=[
            pl.BlockSpec((1, T, _HB, W, C), lambda b, hh: (b, 0, hh, 0, 0)),
            pl.BlockSpec((T, C), lambda b, hh: (0, 0)),
            pl.BlockSpec((_HB, C), lambda b, hh: (hh, 0)),
            pl.BlockSpec((W, C), lambda b, hh: (0, 0)),
        ],
        out_specs=pl.BlockSpec((1, T, _HB, W, C), lambda b, hh: (b, 0, hh, 0, 0)),
        out_shape=jax.ShapeDtypeStruct(x.shape, x.dtype),
        compiler_params=pltpu.CompilerParams(
            dimension_semantics=("arbitrary", "arbitrary"),
        ),
    )(x, T_embed[:T], H_embed[:H], W_embed[:W])
